# baseline (speedup 1.0000x reference)
import functools

import jax
import jax.numpy as jnp
from jax import lax
from jax.experimental import pallas as pl
from jax.experimental.pallas import tpu as pltpu
from jax.experimental.pallas import tpu_sc as plsc

N = 10000
E = 320000
IN_CH = 128
HID = 256
OUT = 128

NC = 2
NS = 16
CH_E = 128
NCH = 2560
E_PAD = NCH * CH_E
N_PAD = 10112
QC = 64
BLK = 400

_MESH = plsc.VectorSubcoreMesh(core_axis_name="c", subcore_axis_name="s")
_SC_PARAMS = pltpu.CompilerParams(use_tc_tiling_on_sc=False)


DEG_CPT = NCH // (NC * NS)


@functools.partial(
    pl.kernel,
    out_type=jax.ShapeDtypeStruct((2 * N,), jnp.float32),
    mesh=_MESH,
    compiler_params=_SC_PARAMS,
    scratch_types=[
        pltpu.VMEM((DEG_CPT, CH_E), jnp.int32),
        pltpu.VMEM((CH_E,), jnp.float32),
        pltpu.VMEM((640,), jnp.float32),
        pltpu.VMEM_SHARED((N_PAD,), jnp.float32),
    ],
)
def _deg_kernel(dst_hbm, deg_out, dstst, ones_v, zeros_v, acc):
    c = lax.axis_index("c")
    s = lax.axis_index("s")
    for k in range(CH_E // 16):
        ones_v[pl.ds(k * 16, 16)] = jnp.ones((16,), jnp.float32)
    for k in range(640 // 16):
        zeros_v[pl.ds(k * 16, 16)] = jnp.zeros((16,), jnp.float32)
    base = (c * NS + s) * DEG_CPT
    pltpu.sync_copy(dst_hbm.at[pl.ds(base, DEG_CPT)], dstst)
    rpt = N_PAD // NS
    pltpu.sync_copy(zeros_v.at[pl.ds(0, rpt)], acc.at[pl.ds(s * rpt, rpt)])
    plsc.subcore_barrier()

    @pl.loop(0, DEG_CPT)
    def _(j):
        pltpu.sync_copy(ones_v, acc.at[dstst.at[j]], add=True)

    plsc.subcore_barrier()

    @pl.when(s < NS - 1)
    def _():
        pltpu.sync_copy(acc.at[pl.ds(s * 624, 624)], zeros_v.at[pl.ds(0, 624)])
        pltpu.sync_copy(zeros_v.at[pl.ds(0, 624)],
                        deg_out.at[pl.ds(c * N + s * 624, 624)])

    @pl.when(s == NS - 1)
    def _():
        pltpu.sync_copy(acc.at[pl.ds(9360, 640)], zeros_v)
        pltpu.sync_copy(zeros_v, deg_out.at[pl.ds(c * N + 9360, 640)])


CPT = NCH // NS


@functools.partial(
    pl.kernel,
    out_type=(jax.ShapeDtypeStruct((N, QC), jnp.float32),
              jax.ShapeDtypeStruct((N, QC), jnp.float32)),
    mesh=_MESH,
    compiler_params=_SC_PARAMS,
    scratch_types=[
        pltpu.VMEM((CPT, CH_E), jnp.int32),
        pltpu.VMEM((CPT, CH_E), jnp.int32),
        [pltpu.VMEM((CH_E, QC), jnp.float32)] * 4,
        pltpu.VMEM_SHARED((N_PAD, QC), jnp.float32),
        [pltpu.SemaphoreType.DMA] * 4,
        [pltpu.SemaphoreType.DMA] * 4,
    ],
)
def _agg(src_hbm, dst_hbm, y0, y1, o0, o1, srcst, dstst, rows, acc,
         gsem, ssem):
    c = lax.axis_index("c")
    s = lax.axis_index("s")
    base = s * CPT
    pltpu.sync_copy(src_hbm.at[pl.ds(base, CPT)], srcst)
    pltpu.sync_copy(dst_hbm.at[pl.ds(base, CPT)], dstst)
    off = jnp.where(s < NS - 1, s * 624, 9360)

    def rows_copy(src_ref, dst_ref, n):
        pltpu.sync_copy(src_ref.at[pl.ds(off, n)], dst_ref.at[pl.ds(off, n)])

    @pl.when(c == 0)
    def _():
        @pl.when(s < NS - 1)
        def _():
            rows_copy(y0, acc, 624)

        @pl.when(s == NS - 1)
        def _():
            rows_copy(y0, acc, 640)

    @pl.when(c == 1)
    def _():
        @pl.when(s < NS - 1)
        def _():
            rows_copy(y1, acc, 624)

        @pl.when(s == NS - 1)
        def _():
            rows_copy(y1, acc, 640)

    plsc.subcore_barrier()

    NB = 4

    def gather(j, b):
        @pl.when(c == 0)
        def _():
            pltpu.async_copy(y0.at[srcst.at[j]], rows[b], gsem[b])

        @pl.when(c == 1)
        def _():
            pltpu.async_copy(y1.at[srcst.at[j]], rows[b], gsem[b])

    def gather_wait(b):
        pltpu.make_async_copy(y0.at[srcst.at[0]], rows[b], gsem[b]).wait()

    for b in range(NB):
        gather(b, b)

    @pl.loop(0, CPT, step=NB)
    def _(j):
        for b in range(NB):
            gather_wait(b)
            pltpu.async_copy(rows[b], acc.at[dstst.at[j + b]], ssem[b],
                             add=True)
        for b in range(NB):
            pltpu.make_async_copy(rows[b], acc.at[dstst.at[0]],
                                  ssem[b]).wait()

            @pl.when(j + NB + b < CPT)
            def _():
                gather(j + NB + b, b)

    plsc.subcore_barrier()

    @pl.when(c == 0)
    def _():
        @pl.when(s < NS - 1)
        def _():
            rows_copy(acc, o0, 624)

        @pl.when(s == NS - 1)
        def _():
            rows_copy(acc, o0, 640)

    @pl.when(c == 1)
    def _():
        @pl.when(s < NS - 1)
        def _():
            rows_copy(acc, o1, 624)

        @pl.when(s == NS - 1)
        def _():
            rows_copy(acc, o1, 640)


def _mm1_body(x_ref, w_ref, d_ref, q0_ref, q1_ref, q2_ref, q3_ref):
    r = lax.rsqrt(d_ref[0] + d_ref[1] + 1.0)
    xw = jnp.dot(x_ref[...], w_ref[...], preferred_element_type=jnp.float32)
    y = xw * r
    q0_ref[...] = y[:, 0 * QC:1 * QC]
    q1_ref[...] = y[:, 1 * QC:2 * QC]
    q2_ref[...] = y[:, 2 * QC:3 * QC]
    q3_ref[...] = y[:, 3 * QC:4 * QC]


def _mid_body(a0_ref, a1_ref, a2_ref, a3_ref, d_ref, b1_ref, w2_ref,
              z0_ref, z1_ref):
    r = lax.rsqrt(d_ref[0] + d_ref[1] + 1.0)
    h = jnp.concatenate(
        [a0_ref[...], a1_ref[...], a2_ref[...], a3_ref[...]], axis=1)
    h = jnp.maximum(h * r + b1_ref[...], 0.0)
    y2 = jnp.dot(h, w2_ref[...], preferred_element_type=jnp.float32) * r
    z0_ref[...] = y2[:, :QC]
    z1_ref[...] = y2[:, QC:]


def _fin_body(g0_ref, g1_ref, d_ref, b2_ref, o_ref):
    r = lax.rsqrt(d_ref[0] + d_ref[1] + 1.0)
    o_ref[...] = (jnp.concatenate([g0_ref[...], g1_ref[...]], axis=1) * r
                  + b2_ref[...])


def _mm1(x, W1, degp):
    return pl.pallas_call(
        _mm1_body,
        grid=(N // BLK,),
        in_specs=[
            pl.BlockSpec((BLK, IN_CH), lambda i: (i, 0)),
            pl.BlockSpec((IN_CH, HID), lambda i: (0, 0)),
            pl.BlockSpec((2, BLK, 1), lambda i: (0, i, 0)),
        ],
        out_specs=tuple(pl.BlockSpec((BLK, QC), lambda i: (i, 0))
                        for _ in range(4)),
        out_shape=tuple(jax.ShapeDtypeStruct((N, QC), jnp.float32)
                        for _ in range(4)),
    )(x, W1, degp)


def _mid(a0, a1, a2, a3, degp, b1r, W2):
    return pl.pallas_call(
        _mid_body,
        grid=(N // BLK,),
        in_specs=[
            pl.BlockSpec((BLK, QC), lambda i: (i, 0)),
            pl.BlockSpec((BLK, QC), lambda i: (i, 0)),
            pl.BlockSpec((BLK, QC), lambda i: (i, 0)),
            pl.BlockSpec((BLK, QC), lambda i: (i, 0)),
            pl.BlockSpec((2, BLK, 1), lambda i: (0, i, 0)),
            pl.BlockSpec((1, HID), lambda i: (0, 0)),
            pl.BlockSpec((HID, OUT), lambda i: (0, 0)),
        ],
        out_specs=(pl.BlockSpec((BLK, QC), lambda i: (i, 0)),
                   pl.BlockSpec((BLK, QC), lambda i: (i, 0))),
        out_shape=(jax.ShapeDtypeStruct((N, QC), jnp.float32),
                   jax.ShapeDtypeStruct((N, QC), jnp.float32)),
    )(a0, a1, a2, a3, degp, b1r, W2)


def _fin(g0, g1, degp, b2r):
    return pl.pallas_call(
        _fin_body,
        grid=(N // BLK,),
        in_specs=[
            pl.BlockSpec((BLK, QC), lambda i: (i, 0)),
            pl.BlockSpec((BLK, QC), lambda i: (i, 0)),
            pl.BlockSpec((2, BLK, 1), lambda i: (0, i, 0)),
            pl.BlockSpec((1, OUT), lambda i: (0, 0)),
        ],
        out_specs=pl.BlockSpec((BLK, OUT), lambda i: (i, 0)),
        out_shape=jax.ShapeDtypeStruct((N, OUT), jnp.float32),
    )(g0, g1, degp, b2r)


def kernel(x, edge_index, W1, b1, W2, b2):
    src = edge_index[0].astype(jnp.int32)
    dst = edge_index[1].astype(jnp.int32)
    pad = E_PAD - E
    srcp = jnp.concatenate([src, jnp.zeros((pad,), jnp.int32)]).reshape(NCH, CH_E)
    dstp = jnp.concatenate(
        [dst, N + (jnp.arange(pad, dtype=jnp.int32) % 16)]).reshape(NCH, CH_E)

    degf = _deg_kernel(dstp)
    degp = degf.reshape(2, N, 1)

    q0, q1, q2, q3 = _mm1(x, W1, degp)
    a0, a1 = _agg(srcp, dstp, q0, q1)
    a2, a3 = _agg(srcp, dstp, q2, q3)
    z0, z1 = _mid(a0, a1, a2, a3, degp, b1.reshape(1, HID), W2)
    g0, g1 = _agg(srcp, dstp, z0, z1)
    return _fin(g0, g1, degp, b2.reshape(1, OUT))


# speedup vs baseline: 1.0080x; 1.0080x over previous
import functools

import jax
import jax.numpy as jnp
from jax import lax
from jax.experimental import pallas as pl
from jax.experimental.pallas import tpu as pltpu
from jax.experimental.pallas import tpu_sc as plsc

N = 10000
E = 320000
IN_CH = 128
HID = 256
OUT = 128

NC = 2
NS = 16
CH_E = 128
NCH = 2560
E_PAD = NCH * CH_E
N_PAD = 10112
QC = 64
BLK = 400

_MESH = plsc.VectorSubcoreMesh(core_axis_name="c", subcore_axis_name="s")
_SC_PARAMS = pltpu.CompilerParams(use_tc_tiling_on_sc=False)


DEG_CPT = NCH // (NC * NS)


@functools.partial(
    pl.kernel,
    out_type=jax.ShapeDtypeStruct((2 * N,), jnp.float32),
    mesh=_MESH,
    compiler_params=_SC_PARAMS,
    scratch_types=[
        pltpu.VMEM((DEG_CPT, CH_E), jnp.int32),
        pltpu.VMEM((CH_E,), jnp.float32),
        pltpu.VMEM((640,), jnp.float32),
        pltpu.VMEM_SHARED((N_PAD,), jnp.float32),
    ],
)
def _deg_kernel(dst_hbm, deg_out, dstst, ones_v, zeros_v, acc):
    c = lax.axis_index("c")
    s = lax.axis_index("s")
    for k in range(CH_E // 16):
        ones_v[pl.ds(k * 16, 16)] = jnp.ones((16,), jnp.float32)
    for k in range(640 // 16):
        zeros_v[pl.ds(k * 16, 16)] = jnp.zeros((16,), jnp.float32)
    base = (c * NS + s) * DEG_CPT
    pltpu.sync_copy(dst_hbm.at[pl.ds(base, DEG_CPT)], dstst)
    rpt = N_PAD // NS
    pltpu.sync_copy(zeros_v.at[pl.ds(0, rpt)], acc.at[pl.ds(s * rpt, rpt)])
    plsc.subcore_barrier()

    @pl.loop(0, DEG_CPT)
    def _(j):
        pltpu.sync_copy(ones_v, acc.at[dstst.at[j]], add=True)

    plsc.subcore_barrier()

    @pl.when(s < NS - 1)
    def _():
        pltpu.sync_copy(acc.at[pl.ds(s * 624, 624)], zeros_v.at[pl.ds(0, 624)])
        pltpu.sync_copy(zeros_v.at[pl.ds(0, 624)],
                        deg_out.at[pl.ds(c * N + s * 624, 624)])

    @pl.when(s == NS - 1)
    def _():
        pltpu.sync_copy(acc.at[pl.ds(9360, 640)], zeros_v)
        pltpu.sync_copy(zeros_v, deg_out.at[pl.ds(c * N + 9360, 640)])


CPT = NCH // NS
NB = 4


def _agg_pass(c, s, srcst, dstst, rows, acc, gsem, ssem, y0, y1, o0, o1):
    off = jnp.where(s < NS - 1, s * 624, 9360)

    def rows_copy(src_ref, dst_ref, n):
        pltpu.sync_copy(src_ref.at[pl.ds(off, n)], dst_ref.at[pl.ds(off, n)])

    def ranged_copy(src_ref, dst_ref):
        @pl.when(s < NS - 1)
        def _():
            rows_copy(src_ref, dst_ref, 624)

        @pl.when(s == NS - 1)
        def _():
            rows_copy(src_ref, dst_ref, 640)

    @pl.when(c == 0)
    def _():
        ranged_copy(y0, acc)

    @pl.when(c == 1)
    def _():
        ranged_copy(y1, acc)

    plsc.subcore_barrier()

    def gather(j, b):
        @pl.when(c == 0)
        def _():
            pltpu.async_copy(y0.at[srcst.at[j]], rows[b], gsem[b])

        @pl.when(c == 1)
        def _():
            pltpu.async_copy(y1.at[srcst.at[j]], rows[b], gsem[b])

    def gather_wait(b):
        pltpu.make_async_copy(y0.at[srcst.at[0]], rows[b], gsem[b]).wait()

    for b in range(NB):
        gather(b, b)

    @pl.loop(0, CPT, step=NB)
    def _(j):
        for b in range(NB):
            gather_wait(b)
            pltpu.async_copy(rows[b], acc.at[dstst.at[j + b]], ssem[b],
                             add=True)
        for b in range(NB):
            pltpu.make_async_copy(rows[b], acc.at[dstst.at[0]],
                                  ssem[b]).wait()

            @pl.when(j + NB + b < CPT)
            def _():
                gather(j + NB + b, b)

    plsc.subcore_barrier()

    @pl.when(c == 0)
    def _():
        ranged_copy(acc, o0)

    @pl.when(c == 1)
    def _():
        ranged_copy(acc, o1)


_AGG_SCRATCH = [
    pltpu.VMEM((CPT, CH_E), jnp.int32),
    pltpu.VMEM((CPT, CH_E), jnp.int32),
    [pltpu.VMEM((CH_E, QC), jnp.float32)] * NB,
    pltpu.VMEM_SHARED((N_PAD, QC), jnp.float32),
    [pltpu.SemaphoreType.DMA] * NB,
    [pltpu.SemaphoreType.DMA] * NB,
]


def _stage_indices(src_hbm, dst_hbm, srcst, dstst, s):
    base = s * CPT
    pltpu.sync_copy(src_hbm.at[pl.ds(base, CPT)], srcst)
    pltpu.sync_copy(dst_hbm.at[pl.ds(base, CPT)], dstst)


@functools.partial(
    pl.kernel,
    out_type=tuple(jax.ShapeDtypeStruct((N, QC), jnp.float32)
                   for _ in range(4)),
    mesh=_MESH,
    compiler_params=_SC_PARAMS,
    scratch_types=_AGG_SCRATCH,
)
def _agg2(src_hbm, dst_hbm, q0, q1, q2, q3, o0, o1, o2, o3, srcst, dstst,
          rows, acc, gsem, ssem):
    c = lax.axis_index("c")
    s = lax.axis_index("s")
    _stage_indices(src_hbm, dst_hbm, srcst, dstst, s)
    _agg_pass(c, s, srcst, dstst, rows, acc, gsem, ssem, q0, q1, o0, o1)
    _agg_pass(c, s, srcst, dstst, rows, acc, gsem, ssem, q2, q3, o2, o3)


@functools.partial(
    pl.kernel,
    out_type=tuple(jax.ShapeDtypeStruct((N, QC), jnp.float32)
                   for _ in range(2)),
    mesh=_MESH,
    compiler_params=_SC_PARAMS,
    scratch_types=_AGG_SCRATCH,
)
def _agg(src_hbm, dst_hbm, y0, y1, o0, o1, srcst, dstst, rows, acc,
         gsem, ssem):
    c = lax.axis_index("c")
    s = lax.axis_index("s")
    _stage_indices(src_hbm, dst_hbm, srcst, dstst, s)
    _agg_pass(c, s, srcst, dstst, rows, acc, gsem, ssem, y0, y1, o0, o1)


def _mm1_body(x_ref, w_ref, d_ref, q0_ref, q1_ref, q2_ref, q3_ref):
    r = lax.rsqrt(d_ref[0] + d_ref[1] + 1.0)
    xw = jnp.dot(x_ref[...], w_ref[...], preferred_element_type=jnp.float32)
    y = xw * r
    q0_ref[...] = y[:, 0 * QC:1 * QC]
    q1_ref[...] = y[:, 1 * QC:2 * QC]
    q2_ref[...] = y[:, 2 * QC:3 * QC]
    q3_ref[...] = y[:, 3 * QC:4 * QC]


def _mid_body(a0_ref, a1_ref, a2_ref, a3_ref, d_ref, b1_ref, w2_ref,
              z0_ref, z1_ref):
    r = lax.rsqrt(d_ref[0] + d_ref[1] + 1.0)
    h = jnp.concatenate(
        [a0_ref[...], a1_ref[...], a2_ref[...], a3_ref[...]], axis=1)
    h = jnp.maximum(h * r + b1_ref[...], 0.0)
    y2 = jnp.dot(h, w2_ref[...], preferred_element_type=jnp.float32) * r
    z0_ref[...] = y2[:, :QC]
    z1_ref[...] = y2[:, QC:]


def _fin_body(g0_ref, g1_ref, d_ref, b2_ref, o_ref):
    r = lax.rsqrt(d_ref[0] + d_ref[1] + 1.0)
    o_ref[...] = (jnp.concatenate([g0_ref[...], g1_ref[...]], axis=1) * r
                  + b2_ref[...])


def _mm1(x, W1, degp):
    return pl.pallas_call(
        _mm1_body,
        grid=(N // BLK,),
        in_specs=[
            pl.BlockSpec((BLK, IN_CH), lambda i: (i, 0)),
            pl.BlockSpec((IN_CH, HID), lambda i: (0, 0)),
            pl.BlockSpec((2, BLK, 1), lambda i: (0, i, 0)),
        ],
        out_specs=tuple(pl.BlockSpec((BLK, QC), lambda i: (i, 0))
                        for _ in range(4)),
        out_shape=tuple(jax.ShapeDtypeStruct((N, QC), jnp.float32)
                        for _ in range(4)),
    )(x, W1, degp)


def _mid(a0, a1, a2, a3, degp, b1r, W2):
    return pl.pallas_call(
        _mid_body,
        grid=(N // BLK,),
        in_specs=[
            pl.BlockSpec((BLK, QC), lambda i: (i, 0)),
            pl.BlockSpec((BLK, QC), lambda i: (i, 0)),
            pl.BlockSpec((BLK, QC), lambda i: (i, 0)),
            pl.BlockSpec((BLK, QC), lambda i: (i, 0)),
            pl.BlockSpec((2, BLK, 1), lambda i: (0, i, 0)),
            pl.BlockSpec((1, HID), lambda i: (0, 0)),
            pl.BlockSpec((HID, OUT), lambda i: (0, 0)),
        ],
        out_specs=(pl.BlockSpec((BLK, QC), lambda i: (i, 0)),
                   pl.BlockSpec((BLK, QC), lambda i: (i, 0))),
        out_shape=(jax.ShapeDtypeStruct((N, QC), jnp.float32),
                   jax.ShapeDtypeStruct((N, QC), jnp.float32)),
    )(a0, a1, a2, a3, degp, b1r, W2)


def _fin(g0, g1, degp, b2r):
    return pl.pallas_call(
        _fin_body,
        grid=(N // BLK,),
        in_specs=[
            pl.BlockSpec((BLK, QC), lambda i: (i, 0)),
            pl.BlockSpec((BLK, QC), lambda i: (i, 0)),
            pl.BlockSpec((2, BLK, 1), lambda i: (0, i, 0)),
            pl.BlockSpec((1, OUT), lambda i: (0, 0)),
        ],
        out_specs=pl.BlockSpec((BLK, OUT), lambda i: (i, 0)),
        out_shape=jax.ShapeDtypeStruct((N, OUT), jnp.float32),
    )(g0, g1, degp, b2r)


def kernel(x, edge_index, W1, b1, W2, b2):
    src = edge_index[0].astype(jnp.int32)
    dst = edge_index[1].astype(jnp.int32)
    pad = E_PAD - E
    srcp = jnp.concatenate([src, jnp.zeros((pad,), jnp.int32)]).reshape(NCH, CH_E)
    dstp = jnp.concatenate(
        [dst, N + (jnp.arange(pad, dtype=jnp.int32) % 16)]).reshape(NCH, CH_E)

    degf = _deg_kernel(dstp)
    degp = degf.reshape(2, N, 1)

    q0, q1, q2, q3 = _mm1(x, W1, degp)
    a0, a1, a2, a3 = _agg2(srcp, dstp, q0, q1, q2, q3)
    z0, z1 = _mid(a0, a1, a2, a3, degp, b1.reshape(1, HID), W2)
    g0, g1 = _agg(srcp, dstp, z0, z1)
    return _fin(g0, g1, degp, b2.reshape(1, OUT))
